# baseline (speedup 1.0000x reference)
import functools

import jax
import jax.numpy as jnp
from jax import lax
from jax.experimental import pallas as pl
from jax.experimental.pallas import tpu as pltpu
from jax.experimental.pallas import tpu_sc as plsc

N = 10000
E = 320000
D = 128
NPAD = 10240
NW = 32
EW = E // NW

EBA = 100
NBA = EW // EBA
EBG = 125
NCHUNK = 8
EC = E // NCHUNK
EWC = EC // NW
NBG = EWC // EBG
ROWS_PER_SUB = NPAD // 16

_MESH = plsc.VectorSubcoreMesh(core_axis_name="c", subcore_axis_name="s")
_SC_PARAMS = pltpu.CompilerParams(
    needs_layout_passes=False, use_tc_tiling_on_sc=False
)


@functools.partial(
    pl.kernel,
    mesh=_MESH,
    out_type=jax.ShapeDtypeStruct((NW, NPAD), jnp.float32),
    compiler_params=_SC_PARAMS,
    scratch_types=[
        pltpu.VMEM((EW,), jnp.int32),
        pltpu.VMEM((NPAD,), jnp.float32),
    ],
)
def _deg_sc(dst_hbm, out_hbm, dst_v, hist_v):
    wid = lax.axis_index("s") * 2 + lax.axis_index("c")
    pltpu.sync_copy(dst_hbm.at[pl.ds(wid * EW, EW)], dst_v)

    def zero_body(i, carry):
        hist_v[pl.ds(i * 16, 16)] = jnp.zeros((16,), jnp.float32)
        return carry

    lax.fori_loop(0, NPAD // 16, zero_body, 0)
    ones = jnp.full((16,), 1.0, jnp.float32)

    def body(g, carry):
        idx = dst_v[pl.ds(g * 16, 16)]
        plsc.addupdate_scatter(hist_v, [idx], ones)
        return carry

    lax.fori_loop(0, EW // 16, body, 0)
    pltpu.sync_copy(hist_v, out_hbm.at[wid])


@functools.partial(
    pl.kernel,
    mesh=_MESH,
    out_type=jax.ShapeDtypeStruct((2, NPAD, D), jnp.float32),
    compiler_params=_SC_PARAMS,
    scratch_types=[
        pltpu.VMEM((NBA, EBA), jnp.int32),
        pltpu.VMEM((NBA, EBA), jnp.int32),
        pltpu.VMEM((2, EBA, D), jnp.float32),
        pltpu.VMEM_SHARED((NPAD, D), jnp.float32),
        pltpu.SemaphoreType.DMA,
        pltpu.SemaphoreType.DMA,
        pltpu.SemaphoreType.DMA,
        pltpu.SemaphoreType.DMA,
    ],
)
def _edge_acc_sc(y_hbm, src_hbm, dst_hbm, out_hbm, src_v, dst_v, rows2_v, acc_sh, g0, g1, s0, s1):
    cid = lax.axis_index("c")
    sid = lax.axis_index("s")
    wid = sid * 2 + cid
    sem_g = (g0, g1)
    sem_s = (s0, s1)
    pltpu.sync_copy(src_hbm.at[pl.ds(wid * NBA, NBA)], src_v)
    pltpu.sync_copy(dst_hbm.at[pl.ds(wid * NBA, NBA)], dst_v)

    def zero_row(i, carry):
        def zcol(j, c2):
            rows2_v[0, i, pl.ds(j * 16, 16)] = jnp.zeros((16,), jnp.float32)
            return c2

        return lax.fori_loop(0, D // 16, zcol, carry)

    lax.fori_loop(0, 64, zero_row, 0)

    def zero_copy(i, carry):
        pltpu.sync_copy(
            rows2_v.at[0, pl.ds(0, 64)],
            acc_sh.at[pl.ds(sid * ROWS_PER_SUB + i * 64, 64)],
        )
        return carry

    lax.fori_loop(0, ROWS_PER_SUB // 64, zero_copy, 0)
    plsc.subcore_barrier()

    def g_start(b, buf):
        pltpu.async_copy(y_hbm.at[src_v.at[b]], rows2_v.at[buf], sem_g[buf])

    def g_wait(b, buf):
        pltpu.make_async_copy(
            y_hbm.at[src_v.at[b]], rows2_v.at[buf], sem_g[buf]
        ).wait()

    def s_start(b, buf):
        pltpu.async_copy(
            rows2_v.at[buf], acc_sh.at[dst_v.at[b]], sem_s[buf], add=True
        )

    def s_wait(b, buf):
        pltpu.make_async_copy(
            rows2_v.at[buf], acc_sh.at[dst_v.at[b]], sem_s[buf]
        ).wait()

    g_start(0, 0)

    def edge_body(i, carry):
        for buf in range(2):
            b = i * 2 + buf
            g_wait(b, buf)

            @pl.when(b + 1 < NBA)
            def _():
                @pl.when(b >= 1)
                def _():
                    s_wait(b - 1, 1 - buf)

                g_start(b + 1, 1 - buf)

            s_start(b, buf)
        return carry

    lax.fori_loop(0, NBA // 2, edge_body, 0)
    s_wait(NBA - 2, 0)
    s_wait(NBA - 1, 1)
    plsc.subcore_barrier()

    def out_body(i, carry):
        r0 = sid * ROWS_PER_SUB + i * 64
        pltpu.sync_copy(acc_sh.at[pl.ds(r0, 64)], rows2_v.at[0, pl.ds(0, 64)])
        pltpu.sync_copy(rows2_v.at[0, pl.ds(0, 64)], out_hbm.at[cid, pl.ds(r0, 64)])
        return carry

    lax.fori_loop(0, ROWS_PER_SUB // 64, out_body, 0)


@functools.partial(
    pl.kernel,
    mesh=_MESH,
    out_type=(
        jax.ShapeDtypeStruct((EC, D), jnp.float32),
        jax.ShapeDtypeStruct((EC, D), jnp.float32),
    ),
    compiler_params=_SC_PARAMS,
    scratch_types=[
        pltpu.VMEM((NBG, EBG), jnp.int32),
        pltpu.VMEM((NBG, EBG), jnp.int32),
        pltpu.VMEM((2, EBG, D), jnp.float32),
        pltpu.VMEM((2, EBG, D), jnp.float32),
        pltpu.SemaphoreType.DMA,
        pltpu.SemaphoreType.DMA,
        pltpu.SemaphoreType.DMA,
        pltpu.SemaphoreType.DMA,
        pltpu.SemaphoreType.DMA,
        pltpu.SemaphoreType.DMA,
        pltpu.SemaphoreType.DMA,
        pltpu.SemaphoreType.DMA,
    ],
)
def _gather_pair_sc(
    z_hbm, src_hbm, dst_hbm, sg_hbm, dg_hbm,
    src_v, dst_v, sb_v, db_v,
    gs0, gs1, gd0, gd1, ws0, ws1, wd0, wd1,
):
    cid = lax.axis_index("c")
    sid = lax.axis_index("s")
    wid = sid * 2 + cid
    sem_gs = (gs0, gs1)
    sem_gd = (gd0, gd1)
    sem_ws = (ws0, ws1)
    sem_wd = (wd0, wd1)
    pltpu.sync_copy(src_hbm.at[pl.ds(wid * NBG, NBG)], src_v)
    pltpu.sync_copy(dst_hbm.at[pl.ds(wid * NBG, NBG)], dst_v)

    def g_start(b, buf):
        pltpu.async_copy(z_hbm.at[src_v.at[b]], sb_v.at[buf], sem_gs[buf])
        pltpu.async_copy(z_hbm.at[dst_v.at[b]], db_v.at[buf], sem_gd[buf])

    def g_wait(b, buf):
        pltpu.make_async_copy(z_hbm.at[src_v.at[b]], sb_v.at[buf], sem_gs[buf]).wait()
        pltpu.make_async_copy(z_hbm.at[dst_v.at[b]], db_v.at[buf], sem_gd[buf]).wait()

    def w_start(b, buf):
        off = wid * EWC + b * EBG
        pltpu.async_copy(sb_v.at[buf], sg_hbm.at[pl.ds(off, EBG)], sem_ws[buf])
        pltpu.async_copy(db_v.at[buf], dg_hbm.at[pl.ds(off, EBG)], sem_wd[buf])

    def w_wait(b, buf):
        off = wid * EWC + b * EBG
        pltpu.make_async_copy(sb_v.at[buf], sg_hbm.at[pl.ds(off, EBG)], sem_ws[buf]).wait()
        pltpu.make_async_copy(db_v.at[buf], dg_hbm.at[pl.ds(off, EBG)], sem_wd[buf]).wait()

    g_start(0, 0)

    def block_body(i, carry):
        for buf in range(2):
            b = i * 2 + buf
            g_wait(b, buf)

            @pl.when(b + 1 < NBG)
            def _():
                @pl.when(b >= 1)
                def _():
                    w_wait(b - 1, 1 - buf)

                g_start(b + 1, 1 - buf)

            w_start(b, buf)
        return carry

    lax.fori_loop(0, NBG // 2, block_body, 0)
    w_wait(NBG - 2, 0)
    w_wait(NBG - 1, 1)


def _dinv_body(parts_ref, o_ref):
    deg = jnp.sum(parts_ref[...], axis=0) + 1.0
    o_ref[...] = jax.lax.rsqrt(deg)


def _dinv_from_parts(parts):
    P = parts.shape[0]
    return pl.pallas_call(
        _dinv_body,
        grid=(NPAD // 1024,),
        in_specs=[pl.BlockSpec((P, 1024), lambda i: (0, i))],
        out_specs=pl.BlockSpec((1024,), lambda i: (i,)),
        out_shape=jax.ShapeDtypeStruct((NPAD,), jnp.float32),
    )(parts)


def _y_body(x_ref, w_ref, dinv_ref, o_ref):
    o_ref[...] = (x_ref[...] @ w_ref[...]) * dinv_ref[...]


def _scaled_matmul(x, w, dinv_col):
    B = 400
    return pl.pallas_call(
        _y_body,
        grid=(N // B,),
        in_specs=[
            pl.BlockSpec((B, D), lambda i: (i, 0)),
            pl.BlockSpec((D, D), lambda i: (0, 0)),
            pl.BlockSpec((B, 1), lambda i: (i, 0)),
        ],
        out_specs=pl.BlockSpec((B, D), lambda i: (i, 0)),
        out_shape=jax.ShapeDtypeStruct((N, D), jnp.float32),
    )(x, w, dinv_col)


def _layer1_body(acc_ref, y_ref, dinv_ref, b_ref, w_ref, o_ref):
    a = acc_ref[0] + acc_ref[1]
    h = jax.nn.relu(dinv_ref[...] * (a + y_ref[...]) + b_ref[...])
    o_ref[...] = (h @ w_ref[...]) * dinv_ref[...]


def _layer1_finish(acc_parts, y1, dinv_col, b1_row, w2):
    B = 400
    return pl.pallas_call(
        _layer1_body,
        grid=(N // B,),
        in_specs=[
            pl.BlockSpec((2, B, D), lambda i: (0, i, 0)),
            pl.BlockSpec((B, D), lambda i: (i, 0)),
            pl.BlockSpec((B, 1), lambda i: (i, 0)),
            pl.BlockSpec((1, D), lambda i: (0, 0)),
            pl.BlockSpec((D, D), lambda i: (0, 0)),
        ],
        out_specs=pl.BlockSpec((B, D), lambda i: (i, 0)),
        out_shape=jax.ShapeDtypeStruct((N, D), jnp.float32),
    )(acc_parts, y1, dinv_col, b1_row, w2)


def _layer2_body(acc_ref, y_ref, dinv_ref, b_ref, o_ref):
    a = acc_ref[0] + acc_ref[1]
    o_ref[...] = dinv_ref[...] * (a + y_ref[...]) + b_ref[...]


def _layer2_finish(acc_parts, y2, dinv_col, b2_row):
    B = 400
    return pl.pallas_call(
        _layer2_body,
        grid=(N // B,),
        in_specs=[
            pl.BlockSpec((2, B, D), lambda i: (0, i, 0)),
            pl.BlockSpec((B, D), lambda i: (i, 0)),
            pl.BlockSpec((B, 1), lambda i: (i, 0)),
            pl.BlockSpec((1, D), lambda i: (0, 0)),
        ],
        out_specs=pl.BlockSpec((B, D), lambda i: (i, 0)),
        out_shape=jax.ShapeDtypeStruct((N, D), jnp.float32),
    )(acc_parts, y2, dinv_col, b2_row)


def _rowdot_body(s_ref, d_ref, o_ref):
    o_ref[...] = jnp.sum(s_ref[...] * d_ref[...], axis=1)


def _rowdot_tc(sg, dg):
    B = 2048
    n = sg.shape[0]
    return pl.pallas_call(
        _rowdot_body,
        grid=(pl.cdiv(n, B),),
        in_specs=[
            pl.BlockSpec((B, D), lambda i: (i, 0)),
            pl.BlockSpec((B, D), lambda i: (i, 0)),
        ],
        out_specs=pl.BlockSpec((B,), lambda i: (i,)),
        out_shape=jax.ShapeDtypeStruct((n,), jnp.float32),
    )(sg, dg)


def kernel(x, edge_index, W1, b1, W2, b2):
    src = edge_index[0]
    dst = edge_index[1]
    src_a = src.reshape(E // EBA, EBA)
    dst_a = dst.reshape(E // EBA, EBA)
    src_g = src.reshape(E // EBG, EBG)
    dst_g = dst.reshape(E // EBG, EBG)
    nrows_c = EC // EBG
    parts = _deg_sc(dst)
    dinv = _dinv_from_parts(parts)
    dinv_col = dinv[:N].reshape(N, 1)
    y1 = _scaled_matmul(x, W1, dinv_col)
    acc1 = _edge_acc_sc(y1, src_a, dst_a)
    y2 = _layer1_finish(acc1, y1, dinv_col, b1.reshape(1, D), W2)
    acc2 = _edge_acc_sc(y2, src_a, dst_a)
    z = _layer2_finish(acc2, y2, dinv_col, b2.reshape(1, D))
    res = []
    for h in range(NCHUNK):
        rows = slice(h * nrows_c, (h + 1) * nrows_c)
        sg, dg = _gather_pair_sc(z, src_g[rows], dst_g[rows])
        res.append(_rowdot_tc(sg, dg))
    return jnp.concatenate(res)


# speedup vs baseline: 1.0159x; 1.0159x over previous
import functools

import jax
import jax.numpy as jnp
from jax import lax
from jax.experimental import pallas as pl
from jax.experimental.pallas import tpu as pltpu
from jax.experimental.pallas import tpu_sc as plsc

N = 10000
E = 320000
D = 128
NPAD = 10240
NW = 32
EW = E // NW

EBA = 100
NBA = EW // EBA
EBG = 125
NCHUNK = 4
EC = E // NCHUNK
EWC = EC // NW
NBG = EWC // EBG
ROWS_PER_SUB = NPAD // 16

_MESH = plsc.VectorSubcoreMesh(core_axis_name="c", subcore_axis_name="s")
_SC_PARAMS = pltpu.CompilerParams(
    needs_layout_passes=False, use_tc_tiling_on_sc=False
)


@functools.partial(
    pl.kernel,
    mesh=_MESH,
    out_type=jax.ShapeDtypeStruct((NW, NPAD), jnp.float32),
    compiler_params=_SC_PARAMS,
    scratch_types=[
        pltpu.VMEM((EW,), jnp.int32),
        pltpu.VMEM((NPAD,), jnp.float32),
    ],
)
def _deg_sc(dst_hbm, out_hbm, dst_v, hist_v):
    wid = lax.axis_index("s") * 2 + lax.axis_index("c")
    pltpu.sync_copy(dst_hbm.at[pl.ds(wid * EW, EW)], dst_v)

    def zero_body(i, carry):
        hist_v[pl.ds(i * 16, 16)] = jnp.zeros((16,), jnp.float32)
        return carry

    lax.fori_loop(0, NPAD // 16, zero_body, 0)
    ones = jnp.full((16,), 1.0, jnp.float32)

    def body(g, carry):
        idx = dst_v[pl.ds(g * 16, 16)]
        plsc.addupdate_scatter(hist_v, [idx], ones)
        return carry

    lax.fori_loop(0, EW // 16, body, 0)
    pltpu.sync_copy(hist_v, out_hbm.at[wid])


@functools.partial(
    pl.kernel,
    mesh=_MESH,
    out_type=jax.ShapeDtypeStruct((2, NPAD, D), jnp.float32),
    compiler_params=_SC_PARAMS,
    scratch_types=[
        pltpu.VMEM((NBA, EBA), jnp.int32),
        pltpu.VMEM((NBA, EBA), jnp.int32),
        pltpu.VMEM((2, EBA, D), jnp.float32),
        pltpu.VMEM_SHARED((NPAD, D), jnp.float32),
        pltpu.SemaphoreType.DMA,
        pltpu.SemaphoreType.DMA,
        pltpu.SemaphoreType.DMA,
        pltpu.SemaphoreType.DMA,
    ],
)
def _edge_acc_sc(y_hbm, src_hbm, dst_hbm, out_hbm, src_v, dst_v, rows2_v, acc_sh, g0, g1, s0, s1):
    cid = lax.axis_index("c")
    sid = lax.axis_index("s")
    wid = sid * 2 + cid
    sem_g = (g0, g1)
    sem_s = (s0, s1)
    pltpu.sync_copy(src_hbm.at[pl.ds(wid * NBA, NBA)], src_v)
    pltpu.sync_copy(dst_hbm.at[pl.ds(wid * NBA, NBA)], dst_v)

    def zero_row(i, carry):
        def zcol(j, c2):
            rows2_v[0, i, pl.ds(j * 16, 16)] = jnp.zeros((16,), jnp.float32)
            return c2

        return lax.fori_loop(0, D // 16, zcol, carry)

    lax.fori_loop(0, 64, zero_row, 0)

    def zero_copy(i, carry):
        pltpu.sync_copy(
            rows2_v.at[0, pl.ds(0, 64)],
            acc_sh.at[pl.ds(sid * ROWS_PER_SUB + i * 64, 64)],
        )
        return carry

    lax.fori_loop(0, ROWS_PER_SUB // 64, zero_copy, 0)
    plsc.subcore_barrier()

    def g_start(b, buf):
        pltpu.async_copy(y_hbm.at[src_v.at[b]], rows2_v.at[buf], sem_g[buf])

    def g_wait(b, buf):
        pltpu.make_async_copy(
            y_hbm.at[src_v.at[b]], rows2_v.at[buf], sem_g[buf]
        ).wait()

    def s_start(b, buf):
        pltpu.async_copy(
            rows2_v.at[buf], acc_sh.at[dst_v.at[b]], sem_s[buf], add=True
        )

    def s_wait(b, buf):
        pltpu.make_async_copy(
            rows2_v.at[buf], acc_sh.at[dst_v.at[b]], sem_s[buf]
        ).wait()

    g_start(0, 0)

    def edge_body(i, carry):
        for buf in range(2):
            b = i * 2 + buf
            g_wait(b, buf)

            @pl.when(b + 1 < NBA)
            def _():
                @pl.when(b >= 1)
                def _():
                    s_wait(b - 1, 1 - buf)

                g_start(b + 1, 1 - buf)

            s_start(b, buf)
        return carry

    lax.fori_loop(0, NBA // 2, edge_body, 0)
    s_wait(NBA - 2, 0)
    s_wait(NBA - 1, 1)
    plsc.subcore_barrier()

    def out_body(i, carry):
        r0 = sid * ROWS_PER_SUB + i * 64
        pltpu.sync_copy(acc_sh.at[pl.ds(r0, 64)], rows2_v.at[0, pl.ds(0, 64)])
        pltpu.sync_copy(rows2_v.at[0, pl.ds(0, 64)], out_hbm.at[cid, pl.ds(r0, 64)])
        return carry

    lax.fori_loop(0, ROWS_PER_SUB // 64, out_body, 0)


@functools.partial(
    pl.kernel,
    mesh=_MESH,
    out_type=(
        jax.ShapeDtypeStruct((EC, D), jnp.float32),
        jax.ShapeDtypeStruct((EC, D), jnp.float32),
    ),
    compiler_params=_SC_PARAMS,
    scratch_types=[
        pltpu.VMEM((NBG, EBG), jnp.int32),
        pltpu.VMEM((NBG, EBG), jnp.int32),
        pltpu.VMEM((2, EBG, D), jnp.float32),
        pltpu.VMEM((2, EBG, D), jnp.float32),
        pltpu.SemaphoreType.DMA,
        pltpu.SemaphoreType.DMA,
        pltpu.SemaphoreType.DMA,
        pltpu.SemaphoreType.DMA,
        pltpu.SemaphoreType.DMA,
        pltpu.SemaphoreType.DMA,
        pltpu.SemaphoreType.DMA,
        pltpu.SemaphoreType.DMA,
    ],
)
def _gather_pair_sc(
    z_hbm, src_hbm, dst_hbm, sg_hbm, dg_hbm,
    src_v, dst_v, sb_v, db_v,
    gs0, gs1, gd0, gd1, ws0, ws1, wd0, wd1,
):
    cid = lax.axis_index("c")
    sid = lax.axis_index("s")
    wid = sid * 2 + cid
    sem_gs = (gs0, gs1)
    sem_gd = (gd0, gd1)
    sem_ws = (ws0, ws1)
    sem_wd = (wd0, wd1)
    pltpu.sync_copy(src_hbm.at[pl.ds(wid * NBG, NBG)], src_v)
    pltpu.sync_copy(dst_hbm.at[pl.ds(wid * NBG, NBG)], dst_v)

    def g_start(b, buf):
        pltpu.async_copy(z_hbm.at[src_v.at[b]], sb_v.at[buf], sem_gs[buf])
        pltpu.async_copy(z_hbm.at[dst_v.at[b]], db_v.at[buf], sem_gd[buf])

    def g_wait(b, buf):
        pltpu.make_async_copy(z_hbm.at[src_v.at[b]], sb_v.at[buf], sem_gs[buf]).wait()
        pltpu.make_async_copy(z_hbm.at[dst_v.at[b]], db_v.at[buf], sem_gd[buf]).wait()

    def w_start(b, buf):
        off = wid * EWC + b * EBG
        pltpu.async_copy(sb_v.at[buf], sg_hbm.at[pl.ds(off, EBG)], sem_ws[buf])
        pltpu.async_copy(db_v.at[buf], dg_hbm.at[pl.ds(off, EBG)], sem_wd[buf])

    def w_wait(b, buf):
        off = wid * EWC + b * EBG
        pltpu.make_async_copy(sb_v.at[buf], sg_hbm.at[pl.ds(off, EBG)], sem_ws[buf]).wait()
        pltpu.make_async_copy(db_v.at[buf], dg_hbm.at[pl.ds(off, EBG)], sem_wd[buf]).wait()

    g_start(0, 0)

    def block_body(i, carry):
        for buf in range(2):
            b = i * 2 + buf
            g_wait(b, buf)

            @pl.when(b + 1 < NBG)
            def _():
                @pl.when(b >= 1)
                def _():
                    w_wait(b - 1, 1 - buf)

                g_start(b + 1, 1 - buf)

            w_start(b, buf)
        return carry

    lax.fori_loop(0, NBG // 2, block_body, 0)
    w_wait(NBG - 2, 0)
    w_wait(NBG - 1, 1)


def _dinv_body(parts_ref, o_ref):
    deg = jnp.sum(parts_ref[...], axis=0) + 1.0
    o_ref[...] = jax.lax.rsqrt(deg)


def _dinv_from_parts(parts):
    P = parts.shape[0]
    return pl.pallas_call(
        _dinv_body,
        grid=(NPAD // 1024,),
        in_specs=[pl.BlockSpec((P, 1024), lambda i: (0, i))],
        out_specs=pl.BlockSpec((1024,), lambda i: (i,)),
        out_shape=jax.ShapeDtypeStruct((NPAD,), jnp.float32),
    )(parts)


def _y_body(x_ref, w_ref, dinv_ref, o_ref):
    o_ref[...] = (x_ref[...] @ w_ref[...]) * dinv_ref[...]


def _scaled_matmul(x, w, dinv_col):
    B = 400
    return pl.pallas_call(
        _y_body,
        grid=(N // B,),
        in_specs=[
            pl.BlockSpec((B, D), lambda i: (i, 0)),
            pl.BlockSpec((D, D), lambda i: (0, 0)),
            pl.BlockSpec((B, 1), lambda i: (i, 0)),
        ],
        out_specs=pl.BlockSpec((B, D), lambda i: (i, 0)),
        out_shape=jax.ShapeDtypeStruct((N, D), jnp.float32),
    )(x, w, dinv_col)


def _layer1_body(acc_ref, y_ref, dinv_ref, b_ref, w_ref, o_ref):
    a = acc_ref[0] + acc_ref[1]
    h = jax.nn.relu(dinv_ref[...] * (a + y_ref[...]) + b_ref[...])
    o_ref[...] = (h @ w_ref[...]) * dinv_ref[...]


def _layer1_finish(acc_parts, y1, dinv_col, b1_row, w2):
    B = 400
    return pl.pallas_call(
        _layer1_body,
        grid=(N // B,),
        in_specs=[
            pl.BlockSpec((2, B, D), lambda i: (0, i, 0)),
            pl.BlockSpec((B, D), lambda i: (i, 0)),
            pl.BlockSpec((B, 1), lambda i: (i, 0)),
            pl.BlockSpec((1, D), lambda i: (0, 0)),
            pl.BlockSpec((D, D), lambda i: (0, 0)),
        ],
        out_specs=pl.BlockSpec((B, D), lambda i: (i, 0)),
        out_shape=jax.ShapeDtypeStruct((N, D), jnp.float32),
    )(acc_parts, y1, dinv_col, b1_row, w2)


def _layer2_body(acc_ref, y_ref, dinv_ref, b_ref, o_ref):
    a = acc_ref[0] + acc_ref[1]
    o_ref[...] = dinv_ref[...] * (a + y_ref[...]) + b_ref[...]


def _layer2_finish(acc_parts, y2, dinv_col, b2_row):
    B = 400
    return pl.pallas_call(
        _layer2_body,
        grid=(N // B,),
        in_specs=[
            pl.BlockSpec((2, B, D), lambda i: (0, i, 0)),
            pl.BlockSpec((B, D), lambda i: (i, 0)),
            pl.BlockSpec((B, 1), lambda i: (i, 0)),
            pl.BlockSpec((1, D), lambda i: (0, 0)),
        ],
        out_specs=pl.BlockSpec((B, D), lambda i: (i, 0)),
        out_shape=jax.ShapeDtypeStruct((N, D), jnp.float32),
    )(acc_parts, y2, dinv_col, b2_row)


def _rowdot_body(s_ref, d_ref, o_ref):
    o_ref[...] = jnp.sum(s_ref[...] * d_ref[...], axis=1)


def _rowdot_tc(sg, dg):
    B = 2048
    n = sg.shape[0]
    return pl.pallas_call(
        _rowdot_body,
        grid=(pl.cdiv(n, B),),
        in_specs=[
            pl.BlockSpec((B, D), lambda i: (i, 0)),
            pl.BlockSpec((B, D), lambda i: (i, 0)),
        ],
        out_specs=pl.BlockSpec((B,), lambda i: (i,)),
        out_shape=jax.ShapeDtypeStruct((n,), jnp.float32),
    )(sg, dg)


def kernel(x, edge_index, W1, b1, W2, b2):
    src = edge_index[0]
    dst = edge_index[1]
    src_a = src.reshape(E // EBA, EBA)
    dst_a = dst.reshape(E // EBA, EBA)
    src_g = src.reshape(E // EBG, EBG)
    dst_g = dst.reshape(E // EBG, EBG)
    nrows_c = EC // EBG
    parts = _deg_sc(dst)
    dinv = _dinv_from_parts(parts)
    dinv_col = dinv[:N].reshape(N, 1)
    y1 = _scaled_matmul(x, W1, dinv_col)
    acc1 = _edge_acc_sc(y1, src_a, dst_a)
    y2 = _layer1_finish(acc1, y1, dinv_col, b1.reshape(1, D), W2)
    acc2 = _edge_acc_sc(y2, src_a, dst_a)
    z = _layer2_finish(acc2, y2, dinv_col, b2.reshape(1, D))
    res = []
    for h in range(NCHUNK):
        rows = slice(h * nrows_c, (h + 1) * nrows_c)
        sg, dg = _gather_pair_sc(z, src_g[rows], dst_g[rows])
        res.append(_rowdot_tc(sg, dg))
    return jnp.concatenate(res)
